# baseline (speedup 1.0000x reference)
import functools

import jax
import jax.numpy as jnp
from jax.experimental import pallas as pl
from jax.experimental.pallas import tpu as pltpu

_N = 8192
_D = 4096
_OUT = 4096
_THRESHOLD = 0.5

_BR = 512
_BM = 512
_BN = 1024


def _stats_kernel(x_ref, energy_ref, rowmean_ref, max_ref, xbf_ref):
    i = pl.program_id(0)
    xb = x_ref[...]
    absmean = jnp.mean(jnp.abs(xb), axis=1, keepdims=True)
    energy_ref[...] = absmean
    rowmean_ref[...] = jnp.mean(xb, axis=1, keepdims=True)
    xbf_ref[...] = xb.astype(jnp.bfloat16)
    bmax = jnp.max(absmean, axis=None, keepdims=True)

    @pl.when(i == 0)
    def _():
        max_ref[...] = bmax

    @pl.when(i > 0)
    def _():
        max_ref[...] = jnp.maximum(max_ref[...], bmax)


def _mm_kernel(x_ref, w_ref, b_ref, energy_ref, max_ref, rowmean_ref, o_ref):
    acc = jax.lax.dot_general(
        x_ref[...], w_ref[...],
        dimension_numbers=(((1,), (1,)), ((), ())),
        preferred_element_type=jnp.float32,
    )
    res = acc + b_ref[0, :][None, :]
    thresh = _THRESHOLD * (max_ref[...] + 1e-8)
    active = energy_ref[...] >= thresh
    o_ref[...] = jnp.where(active, res, rowmean_ref[...])


@jax.jit
def kernel(x, W, b):
    n, d = x.shape
    out_f = W.shape[0]

    energy, rowmean, emax, x_bf = pl.pallas_call(
        _stats_kernel,
        grid=(n // _BR,),
        in_specs=[pl.BlockSpec((_BR, d), lambda i: (i, 0))],
        out_specs=[
            pl.BlockSpec((_BR, 1), lambda i: (i, 0)),
            pl.BlockSpec((_BR, 1), lambda i: (i, 0)),
            pl.BlockSpec((1, 1), lambda i: (0, 0)),
            pl.BlockSpec((_BR, d), lambda i: (i, 0)),
        ],
        out_shape=[
            jax.ShapeDtypeStruct((n, 1), jnp.float32),
            jax.ShapeDtypeStruct((n, 1), jnp.float32),
            jax.ShapeDtypeStruct((1, 1), jnp.float32),
            jax.ShapeDtypeStruct((n, d), jnp.bfloat16),
        ],
    )(x)

    w_bf = W.astype(jnp.bfloat16)
    b2 = b.reshape(1, out_f)

    out = pl.pallas_call(
        _mm_kernel,
        grid=(out_f // _BN, n // _BM),
        in_specs=[
            pl.BlockSpec((_BM, d), lambda j, i: (i, 0)),
            pl.BlockSpec((_BN, d), lambda j, i: (j, 0)),
            pl.BlockSpec((1, _BN), lambda j, i: (0, j)),
            pl.BlockSpec((_BM, 1), lambda j, i: (i, 0)),
            pl.BlockSpec((1, 1), lambda j, i: (0, 0)),
            pl.BlockSpec((_BM, 1), lambda j, i: (i, 0)),
        ],
        out_specs=pl.BlockSpec((_BM, _BN), lambda j, i: (i, j)),
        out_shape=jax.ShapeDtypeStruct((n, out_f), jnp.float32),
        compiler_params=pltpu.CompilerParams(
            dimension_semantics=("parallel", "parallel"),
        ),
    )(x_bf, w_bf, b2, energy, emax, rowmean)
    return out


# speedup vs baseline: 1.0948x; 1.0948x over previous
import functools

import jax
import jax.numpy as jnp
from jax.experimental import pallas as pl
from jax.experimental.pallas import tpu as pltpu

_N = 8192
_D = 4096
_OUT = 4096
_THRESHOLD = 0.5

_BR = 512
_BM = 512
_BN = 1024


def _stats_kernel(x_ref, energy_ref, rowmean_ref, max_ref):
    i = pl.program_id(0)
    xb = x_ref[...]
    absmean = jnp.mean(jnp.abs(xb), axis=1, keepdims=True)
    energy_ref[...] = absmean
    rowmean_ref[...] = jnp.mean(xb, axis=1, keepdims=True)
    bmax = jnp.max(absmean, axis=None, keepdims=True)

    @pl.when(i == 0)
    def _():
        max_ref[...] = bmax

    @pl.when(i > 0)
    def _():
        max_ref[...] = jnp.maximum(max_ref[...], bmax)


def _mm_kernel(x_ref, w_ref, b_ref, energy_ref, max_ref, rowmean_ref, o_ref):
    acc = jax.lax.dot_general(
        x_ref[...].astype(jnp.bfloat16), w_ref[...].astype(jnp.bfloat16),
        dimension_numbers=(((1,), (1,)), ((), ())),
        preferred_element_type=jnp.float32,
    )
    res = acc + b_ref[0, :][None, :]
    thresh = _THRESHOLD * (max_ref[...] + 1e-8)
    active = energy_ref[...] >= thresh
    o_ref[...] = jnp.where(active, res, rowmean_ref[...])


@jax.jit
def kernel(x, W, b):
    n, d = x.shape
    out_f = W.shape[0]

    energy, rowmean, emax = pl.pallas_call(
        _stats_kernel,
        grid=(n // _BR,),
        in_specs=[pl.BlockSpec((_BR, d), lambda i: (i, 0))],
        out_specs=[
            pl.BlockSpec((_BR, 1), lambda i: (i, 0)),
            pl.BlockSpec((_BR, 1), lambda i: (i, 0)),
            pl.BlockSpec((1, 1), lambda i: (0, 0)),
        ],
        out_shape=[
            jax.ShapeDtypeStruct((n, 1), jnp.float32),
            jax.ShapeDtypeStruct((n, 1), jnp.float32),
            jax.ShapeDtypeStruct((1, 1), jnp.float32),
        ],
    )(x)

    b2 = b.reshape(1, out_f)

    out = pl.pallas_call(
        _mm_kernel,
        grid=(out_f // _BN, n // _BM),
        in_specs=[
            pl.BlockSpec((_BM, d), lambda j, i: (i, 0)),
            pl.BlockSpec((_BN, d), lambda j, i: (j, 0)),
            pl.BlockSpec((1, _BN), lambda j, i: (0, j)),
            pl.BlockSpec((_BM, 1), lambda j, i: (i, 0)),
            pl.BlockSpec((1, 1), lambda j, i: (0, 0)),
            pl.BlockSpec((_BM, 1), lambda j, i: (i, 0)),
        ],
        out_specs=pl.BlockSpec((_BM, _BN), lambda j, i: (i, j)),
        out_shape=jax.ShapeDtypeStruct((n, out_f), jnp.float32),
        compiler_params=pltpu.CompilerParams(
            dimension_semantics=("arbitrary", "arbitrary"),
        ),
    )(x, W, b2, energy, emax, rowmean)
    return out
